# Initial kernel scaffold; baseline (speedup 1.0000x reference)
import jax
import jax.numpy as jnp
from jax.experimental import pallas as pl


def kernel(x, bias):
    raise NotImplementedError("write your pallas kernel here")


# speedup vs baseline: 24.8259x; 24.8259x over previous
import jax
import jax.numpy as jnp
from jax.experimental import pallas as pl
from jax.experimental.pallas import tpu as pltpu

_FIN = 0x7F800000
_NPASS = 8


def _sparsity() -> float:
    s_i, s_f = 0.5, 0.9
    return s_f + (s_i - s_f) * (1.0 - 0.0) ** 3


def _mid(lo, hi, j):
    size = hi - lo
    return lo + (size >> 4) * j + (((size & 15) * j) >> 4)


def _thresh_body(nchunks, k, x_ref, t_ref, bounds, counts):
    p = pl.program_id(0)
    c = pl.program_id(1)

    @pl.when((p == 0) & (c == 0))
    def _():
        bounds[0] = 0
        bounds[1] = _FIN

    @pl.when(c == 0)
    def _():
        for j in range(1, 16):
            counts[j] = 0

    lo = bounds[0]
    hi = bounds[1]
    bits = jax.lax.bitcast_convert_type(x_ref[...], jnp.int32) & jnp.int32(0x7FFFFFFF)
    for j in range(1, 16):
        counts[j] += jnp.sum(bits >= _mid(lo, hi, j))

    @pl.when(c == nchunks - 1)
    def _():
        jstar = jnp.int32(0)
        for j in range(1, 16):
            jstar += (counts[j] >= k).astype(jnp.int32)
        bounds[0] = _mid(lo, hi, jstar)
        bounds[1] = _mid(lo, hi, jstar + 1)
        t_ref[0] = bounds[0]


def _mask_body(t_ref, x_ref, o_ref):
    t = t_ref[0]
    xv = x_ref[...]
    bits = jax.lax.bitcast_convert_type(xv, jnp.int32) & jnp.int32(0x7FFFFFFF)
    o_ref[...] = jnp.where(bits >= t, xv, 0.0)


def kernel(x, bias):
    rows, cols = x.shape
    numel = rows * cols
    k = max(1, int(round(numel * (1.0 - _sparsity()))))
    nchunks = 8
    blk = rows // nchunks

    tbits = pl.pallas_call(
        lambda x_ref, t_ref, bounds, counts: _thresh_body(
            nchunks, k, x_ref, t_ref, bounds, counts
        ),
        grid=(_NPASS, nchunks),
        in_specs=[pl.BlockSpec((blk, cols), lambda p, c: (c, 0))],
        out_specs=pl.BlockSpec(memory_space=pltpu.SMEM),
        out_shape=jax.ShapeDtypeStruct((1,), jnp.int32),
        scratch_shapes=[pltpu.SMEM((2,), jnp.int32), pltpu.SMEM((16,), jnp.int32)],
    )(x)

    masked = pl.pallas_call(
        _mask_body,
        grid=(nchunks,),
        in_specs=[
            pl.BlockSpec(memory_space=pltpu.SMEM),
            pl.BlockSpec((blk, cols), lambda c: (c, 0)),
        ],
        out_specs=pl.BlockSpec((blk, cols), lambda c: (c, 0)),
        out_shape=jax.ShapeDtypeStruct((rows, cols), jnp.float32),
    )(tbits, x)

    return (masked, bias)
